# baseline (speedup 1.0000x reference)
import functools

import jax
import jax.numpy as jnp
from jax import lax
from jax.experimental import pallas as pl
from jax.experimental.pallas import tpu as pltpu
from jax.experimental.pallas import tpu_sc as plsc

N = 106496
NUM_FIELDS = 100
D = 64
NUM_SCALES = 5
R = NUM_SCALES + 1
BLK = R * D
NW = 32
CHUNK = N // NW
SUB = 256
CPAD = SUB + 1
NSUB = CHUNK // SUB
GROUPS = SUB // 16
LANES = 16

_SF = [0.01, 0.1, 1.0, 10.0, 100.0]


def _body(
    table_hbm, ids_hbm, vals_hbm, out_hbm,
    table_v, ids_v, vals_v, col_a, col_b, sem_a, sem_b,
):
    wid = lax.axis_index("s") * 2 + lax.axis_index("c")
    base_tok = wid * CHUNK
    pltpu.sync_copy(table_hbm, table_v)
    pltpu.sync_copy(ids_hbm.at[pl.ds(base_tok, CHUNK)], ids_v)
    pltpu.sync_copy(vals_hbm.at[pl.ds(base_tok, CHUNK)], vals_v)
    lane = lax.iota(jnp.int32, LANES)
    rowvec = [jnp.int32(j * LANES) + lane for j in range(D // LANES)]

    def grp_body(s, buf, g, _):
        t0 = s * SUB + g * LANES
        fvec = ids_v[pl.ds(t0, LANES)]
        vvec = vals_v[pl.ds(t0, LANES)]
        ws = []
        for r in range(NUM_SCALES):
            x = vvec * (2.0 * _SF[r])
            ws.append(1.0 - 2.0 / (jnp.exp(x) + 1.0))
        for t in range(LANES):
            base = fvec[t] * (BLK // 2)
            colv = jnp.broadcast_to(g * LANES + t, (LANES,))
            rows = [
                plsc.bitcast(table_v[pl.ds(base + k * LANES, LANES)], jnp.bfloat16)
                for k in range(BLK // 32)
            ]
            wbfs = []
            for r in range(NUM_SCALES):
                wf = jnp.broadcast_to(ws[r][t], (LANES,))
                wbfs.append(plsc.pack(wf, wf, format=plsc.PackFormat.INTERLEAVED))
            for h in range(2):
                acc = rows[NUM_SCALES * 2 + h]
                for r in range(NUM_SCALES):
                    acc = acc + wbfs[r] * rows[r * 2 + h]
                a, b = plsc.unpack(acc, format=plsc.PackFormat.INTERLEAVED)
                plsc.store_scatter(buf, [rowvec[2 * h], colv], a)
                plsc.store_scatter(buf, [rowvec[2 * h + 1], colv], b)
        return _

    def sub_body(s, _):
        p = s % 2
        dst = out_hbm.at[:, pl.ds(base_tok + s * SUB, SUB)]

        @pl.when(jnp.logical_and(s >= 2, p == 0))
        def _wait_a():
            pltpu.make_async_copy(col_a.at[:, pl.ds(0, SUB)], dst, sem_a).wait()

        @pl.when(jnp.logical_and(s >= 2, p == 1))
        def _wait_b():
            pltpu.make_async_copy(col_b.at[:, pl.ds(0, SUB)], dst, sem_b).wait()

        @pl.when(p == 0)
        def _run_a():
            lax.fori_loop(0, GROUPS, functools.partial(grp_body, s, col_a), 0)
            pltpu.async_copy(col_a.at[:, pl.ds(0, SUB)], dst, sem_a)

        @pl.when(p == 1)
        def _run_b():
            lax.fori_loop(0, GROUPS, functools.partial(grp_body, s, col_b), 0)
            pltpu.async_copy(col_b.at[:, pl.ds(0, SUB)], dst, sem_b)

        return _

    lax.fori_loop(0, NSUB, sub_body, 0)
    tail = out_hbm.at[:, pl.ds(base_tok, SUB)]
    pltpu.make_async_copy(col_a.at[:, pl.ds(0, SUB)], tail, sem_a).wait()
    pltpu.make_async_copy(col_b.at[:, pl.ds(0, SUB)], tail, sem_b).wait()


def kernel(field_ids, values, field_table, scale_table):
    scale2 = scale_table.reshape(NUM_FIELDS, NUM_SCALES * D)
    combined = jnp.concatenate([scale2, field_table], axis=1)
    t5 = combined.reshape(NUM_FIELDS, R, 2, 2, LANES)
    table_bf = t5.transpose(0, 1, 2, 4, 3).astype(jnp.bfloat16).reshape(-1, 2)
    u16 = jax.lax.bitcast_convert_type(table_bf, jnp.uint16)
    table_u = u16[:, 0].astype(jnp.uint32) | (u16[:, 1].astype(jnp.uint32) << 16)
    ids = field_ids.astype(jnp.int32)
    mesh = plsc.VectorSubcoreMesh(core_axis_name="c", subcore_axis_name="s")
    k = pl.kernel(
        _body,
        out_type=jax.ShapeDtypeStruct((D, N), jnp.float32),
        mesh=mesh,
        compiler_params=pltpu.CompilerParams(
            needs_layout_passes=False, use_tc_tiling_on_sc=True
        ),
        scratch_types=[
            pltpu.VMEM((NUM_FIELDS * BLK // 2,), jnp.uint32),
            pltpu.VMEM((CHUNK,), jnp.int32),
            pltpu.VMEM((CHUNK,), jnp.float32),
            pltpu.VMEM((D, CPAD), jnp.float32),
            pltpu.VMEM((D, CPAD), jnp.float32),
            pltpu.SemaphoreType.DMA,
            pltpu.SemaphoreType.DMA,
        ],
    )
    return k(table_u, ids, values).T


# speedup vs baseline: 2.0118x; 2.0118x over previous
import functools

import jax
import jax.numpy as jnp
from jax import lax
from jax.experimental import pallas as pl
from jax.experimental.pallas import tpu as pltpu
from jax.experimental.pallas import tpu_sc as plsc

N = 106496
NUM_FIELDS = 100
D = 64
NUM_SCALES = 5
R = NUM_SCALES + 1
BLK = R * D
NW = 32
CHUNK = N // NW
SUB = 256
CPAD = SUB + 1
NSUB = CHUNK // SUB
GROUPS = SUB // 16
LANES = 16

_SF = [0.01, 0.1, 1.0, 10.0, 100.0]


def _body(
    table_hbm, ids_hbm, vals_hbm, out_hbm,
    table_v, ids_v, vals_v, col_v, out_t, sem_a, sem_b,
):
    wid = lax.axis_index("s") * 2 + lax.axis_index("c")
    base_tok = wid * CHUNK
    pltpu.sync_copy(table_hbm, table_v)
    pltpu.sync_copy(ids_hbm.at[pl.ds(base_tok, CHUNK)], ids_v)
    pltpu.sync_copy(vals_hbm.at[pl.ds(base_tok, CHUNK)], vals_v)
    lane = lax.iota(jnp.int32, LANES)
    rowbase = [(jnp.int32(j * LANES) + lane) * CPAD for j in range(D // LANES)]

    def weights_for(t0):
        fvec = ids_v[pl.ds(t0, LANES)]
        vvec = vals_v[pl.ds(t0, LANES)]
        ws = []
        for r in range(NUM_SCALES):
            x = vvec * (2.0 * _SF[r])
            ws.append(1.0 - 2.0 / (jnp.exp(x) + 1.0))
        return (fvec, *ws)

    def grp_body(s, g, carry):
        fvec = carry[0]
        ws = carry[1:]
        t0n = jnp.minimum(s * SUB + (g + 1) * LANES, CHUNK - LANES)
        nxt = weights_for(t0n)
        for t in range(LANES):
            base = fvec[t] * (BLK // 2)
            col = g * LANES + t
            rows = [
                plsc.bitcast(table_v[pl.ds(base + k * LANES, LANES)], jnp.bfloat16)
                for k in range(BLK // 32)
            ]
            wbfs = []
            for r in range(NUM_SCALES):
                wf = jnp.broadcast_to(ws[r][t], (LANES,))
                wbfs.append(plsc.pack(wf, wf, format=plsc.PackFormat.INTERLEAVED))
            for h in range(2):
                acc = rows[NUM_SCALES * 2 + h]
                for r in range(NUM_SCALES):
                    acc = acc + wbfs[r] * rows[r * 2 + h]
                a, b = plsc.unpack(acc, format=plsc.PackFormat.INTERLEAVED)
                plsc.store_scatter(col_v, [rowbase[2 * h] + col], a)
                plsc.store_scatter(col_v, [rowbase[2 * h + 1] + col], b)
        return nxt

    def sub_body(s, _):
        p = s % 2
        dst = out_hbm.at[:, pl.ds(base_tok + s * SUB, SUB)]

        @pl.when(jnp.logical_and(s >= 2, p == 0))
        def _wait_a():
            pltpu.make_async_copy(out_t.at[0], dst, sem_a).wait()

        @pl.when(jnp.logical_and(s >= 2, p == 1))
        def _wait_b():
            pltpu.make_async_copy(out_t.at[1], dst, sem_b).wait()

        lax.fori_loop(0, GROUPS, functools.partial(grp_body, s), weights_for(s * SUB))

        @pl.when(p == 0)
        def _flush_a():
            for d in range(D):
                for k in range(SUB // LANES):
                    out_t[0, d, pl.ds(k * LANES, LANES)] = col_v[
                        pl.ds(d * CPAD + k * LANES, LANES)
                    ]
            pltpu.async_copy(out_t.at[0], dst, sem_a)

        @pl.when(p == 1)
        def _flush_b():
            for d in range(D):
                for k in range(SUB // LANES):
                    out_t[1, d, pl.ds(k * LANES, LANES)] = col_v[
                        pl.ds(d * CPAD + k * LANES, LANES)
                    ]
            pltpu.async_copy(out_t.at[1], dst, sem_b)

        return _

    lax.fori_loop(0, NSUB, sub_body, 0)
    tail = out_hbm.at[:, pl.ds(base_tok, SUB)]
    pltpu.make_async_copy(out_t.at[0], tail, sem_a).wait()
    pltpu.make_async_copy(out_t.at[1], tail, sem_b).wait()


def kernel(field_ids, values, field_table, scale_table):
    scale2 = scale_table.reshape(NUM_FIELDS, NUM_SCALES * D)
    combined = jnp.concatenate([scale2, field_table], axis=1)
    t5 = combined.reshape(NUM_FIELDS, R, 2, 2, LANES)
    table_bf = t5.transpose(0, 1, 2, 4, 3).astype(jnp.bfloat16).reshape(-1, 2)
    u16 = jax.lax.bitcast_convert_type(table_bf, jnp.uint16)
    table_u = u16[:, 0].astype(jnp.uint32) | (u16[:, 1].astype(jnp.uint32) << 16)
    ids = field_ids.astype(jnp.int32)
    mesh = plsc.VectorSubcoreMesh(core_axis_name="c", subcore_axis_name="s")
    k = pl.kernel(
        _body,
        out_type=jax.ShapeDtypeStruct((D, N), jnp.float32),
        mesh=mesh,
        compiler_params=pltpu.CompilerParams(
            needs_layout_passes=False, use_tc_tiling_on_sc=True
        ),
        scratch_types=[
            pltpu.VMEM((NUM_FIELDS * BLK // 2,), jnp.uint32),
            pltpu.VMEM((CHUNK,), jnp.int32),
            pltpu.VMEM((CHUNK,), jnp.float32),
            pltpu.VMEM((D * CPAD,), jnp.float32),
            pltpu.VMEM((2, D, SUB), jnp.float32),
            pltpu.SemaphoreType.DMA,
            pltpu.SemaphoreType.DMA,
        ],
    )
    return k(table_u, ids, values).T
